# baseline (speedup 1.0000x reference)
import jax
import jax.numpy as jnp
from jax.experimental import pallas as pl

_STRIDE = 32


def kernel(x):
    b, h, s, d = x.shape
    k = s // _STRIDE
    x2 = x.reshape(b * h, k, _STRIDE * d)

    grid_n = 8
    g = (b * h) // grid_n

    def body(in_ref, out_ref):
        out_ref[...] = in_ref[...]

    out = pl.pallas_call(
        body,
        grid=(grid_n,),
        in_specs=[pl.BlockSpec((g, k, d), lambda i: (i, 0, 0))],
        out_specs=pl.BlockSpec((g, k, d), lambda i: (i, 0, 0)),
        out_shape=jax.ShapeDtypeStruct((b * h, k, d), x.dtype),
    )(x2)
    return out.reshape(b, h, k, d)
